# Initial kernel scaffold; baseline (speedup 1.0000x reference)
import jax
import jax.numpy as jnp
from jax.experimental import pallas as pl


def kernel(x, edge_index, Wrel0, brel0, Wroot0, Wrel1, brel1, Wroot1, Wrel2, brel2, Wroot2, Wlin, blin):
    raise NotImplementedError("write your pallas kernel here")


# speedup vs baseline: 4.6329x; 4.6329x over previous
import functools

import jax
import jax.numpy as jnp
from jax import lax
from jax.experimental import pallas as pl
from jax.experimental.pallas import tpu as pltpu
from jax.experimental.pallas import tpu_sc as plsc

N = 10000
E = 320000
D = 128
Q = 32
C = 40

NC = 2
NS = 16

G = 128
NG_TOT = 2560
E_PAD = NG_TOT * G
NG = NG_TOT // NS
NBUF = 4

N_PAD = 10240
TRASH = N
ROWS_PER_TILE = N_PAD // NS


def _make_agg():
    mesh = plsc.VectorSubcoreMesh(core_axis_name="c", subcore_axis_name="s")

    @functools.partial(
        pl.kernel,
        out_type=jax.ShapeDtypeStruct((NC, N_PAD, Q), jnp.float32),
        mesh=mesh,
        scratch_types=[
            pltpu.VMEM((NG, G), jnp.int32),
            pltpu.VMEM((NG, G), jnp.int32),
            pltpu.VMEM((NBUF, G, Q), jnp.float32),
            pltpu.VMEM_SHARED((N_PAD, Q), jnp.float32),
            pltpu.SemaphoreType.DMA,
            pltpu.SemaphoreType.DMA,
            pltpu.SemaphoreType.DMA,
            pltpu.SemaphoreType.DMA,
        ],
        compiler_params=pltpu.CompilerParams(use_tc_tiling_on_sc=False),
    )
    def agg(src_hbm, dst_hbm, hq_hbm, out_hbm, src_v, dst_v, rows_v, acc_sh,
            sem0, sem1, sem2, sem3):
        sems = (sem0, sem1, sem2, sem3)
        c = lax.axis_index("c")
        s = lax.axis_index("s")

        pltpu.sync_copy(src_hbm.at[pl.ds(s * NG, NG)], src_v)
        pltpu.sync_copy(dst_hbm.at[pl.ds(s * NG, NG)], dst_v)

        def _zrow(i, carry):
            for j in range(Q // 16):
                rows_v[0, i, pl.ds(j * 16, 16)] = jnp.zeros((16,), jnp.float32)
            return carry
        lax.fori_loop(0, G, _zrow, 0)
        for k in range(ROWS_PER_TILE // G):
            pltpu.sync_copy(rows_v.at[0],
                            acc_sh.at[pl.ds(s * ROWS_PER_TILE + k * G, G)])
        plsc.subcore_barrier()

        table = hq_hbm.at[c]
        for b in range(NBUF):
            pltpu.async_copy(table.at[src_v.at[b]], rows_v.at[b], sems[b])

        def _outer(i, carry):
            t = i * NBUF
            for b in range(NBUF):
                g = t + b
                pltpu.make_async_copy(table.at[src_v.at[b]], rows_v.at[b],
                                      sems[b]).wait()
                pltpu.sync_copy(rows_v.at[b], acc_sh.at[dst_v.at[g]], add=True)

                @pl.when(g + NBUF < NG)
                def _():
                    pltpu.async_copy(table.at[src_v.at[g + NBUF]],
                                     rows_v.at[b], sems[b])
            return carry
        lax.fori_loop(0, NG // NBUF, _outer, 0)
        plsc.subcore_barrier()

        for k in range(ROWS_PER_TILE // G):
            r0 = s * ROWS_PER_TILE + k * G
            pltpu.sync_copy(acc_sh.at[pl.ds(r0, G)],
                            out_hbm.at[c].at[pl.ds(r0, G)])

    return agg


_agg = _make_agg()

BS = 1000
NBLK = N // BS


def _dense_mid(pA, pB, hA, hB, Wrel, brel, Wroot):
    def body(pa_ref, pb_ref, ha_ref, hb_ref, wrel_ref, brel_ref, wroot_ref,
             oa_ref, ob_ref):
        agg = jnp.concatenate(
            [pa_ref[0], pa_ref[1], pb_ref[0], pb_ref[1]], axis=1)
        hh = jnp.concatenate(
            [ha_ref[0], ha_ref[1], hb_ref[0], hb_ref[1]], axis=1)
        val = jnp.maximum(
            jnp.dot(agg, wrel_ref[...], preferred_element_type=jnp.float32)
            + jnp.dot(hh, wroot_ref[...], preferred_element_type=jnp.float32)
            + brel_ref[...], 0.0)
        oa_ref[0] = val[:, 0 * Q:1 * Q]
        oa_ref[1] = val[:, 1 * Q:2 * Q]
        ob_ref[0] = val[:, 2 * Q:3 * Q]
        ob_ref[1] = val[:, 3 * Q:4 * Q]

    blk = pl.BlockSpec((NC, BS, Q), lambda i: (0, i, 0))
    return pl.pallas_call(
        body,
        grid=(NBLK,),
        in_specs=[
            blk, blk, blk, blk,
            pl.BlockSpec((D, D), lambda i: (0, 0)),
            pl.BlockSpec((1, D), lambda i: (0, 0)),
            pl.BlockSpec((D, D), lambda i: (0, 0)),
        ],
        out_specs=[blk, blk],
        out_shape=[jax.ShapeDtypeStruct((NC, N, Q), jnp.float32),
                   jax.ShapeDtypeStruct((NC, N, Q), jnp.float32)],
    )(pA, pB, hA, hB, Wrel, brel.reshape(1, D), Wroot)


def _dense_last(pA, pB, hA, hB, Wrel, brel, Wroot, Wlin, blin):
    def body(pa_ref, pb_ref, ha_ref, hb_ref, wrel_ref, brel_ref, wroot_ref,
             wlin_ref, blin_ref, o_ref):
        agg = jnp.concatenate(
            [pa_ref[0], pa_ref[1], pb_ref[0], pb_ref[1]], axis=1)
        hh = jnp.concatenate(
            [ha_ref[0], ha_ref[1], hb_ref[0], hb_ref[1]], axis=1)
        h3 = jnp.maximum(
            jnp.dot(agg, wrel_ref[...], preferred_element_type=jnp.float32)
            + jnp.dot(hh, wroot_ref[...], preferred_element_type=jnp.float32)
            + brel_ref[...], 0.0)
        o_ref[...] = (jnp.dot(h3, wlin_ref[...],
                              preferred_element_type=jnp.float32)
                      + blin_ref[...])

    blk = pl.BlockSpec((NC, BS, Q), lambda i: (0, i, 0))
    return pl.pallas_call(
        body,
        grid=(NBLK,),
        in_specs=[
            blk, blk, blk, blk,
            pl.BlockSpec((D, D), lambda i: (0, 0)),
            pl.BlockSpec((1, D), lambda i: (0, 0)),
            pl.BlockSpec((D, D), lambda i: (0, 0)),
            pl.BlockSpec((D, C), lambda i: (0, 0)),
            pl.BlockSpec((1, C), lambda i: (0, 0)),
        ],
        out_specs=pl.BlockSpec((BS, C), lambda i: (i, 0)),
        out_shape=jax.ShapeDtypeStruct((N, C), jnp.float32),
    )(pA, pB, hA, hB, Wrel, brel.reshape(1, D), Wroot, Wlin,
      blin.reshape(1, C))


def kernel(x, edge_index, Wrel0, brel0, Wroot0, Wrel1, brel1, Wroot1,
           Wrel2, brel2, Wroot2, Wlin, blin):
    pad = E_PAD - E
    src2 = jnp.concatenate(
        [edge_index[0], jnp.zeros((pad,), jnp.int32)]).reshape(NG_TOT, G)
    dst2 = jnp.concatenate(
        [edge_index[1], jnp.full((pad,), TRASH, jnp.int32)]).reshape(NG_TOT, G)

    x4 = x.reshape(N, 2 * NC, Q).transpose(1, 0, 2)
    hA, hB = x4[:NC], x4[NC:]
    for (Wrel, brel, Wroot) in ((Wrel0, brel0, Wroot0), (Wrel1, brel1, Wroot1)):
        pA = _agg(src2, dst2, hA)
        pB = _agg(src2, dst2, hB)
        hA, hB = _dense_mid(pA, pB, hA, hB, Wrel, brel, Wroot)
    pA = _agg(src2, dst2, hA)
    pB = _agg(src2, dst2, hB)
    return _dense_last(pA, pB, hA, hB, Wrel2, brel2, Wroot2, Wlin, blin)
